# Initial kernel scaffold; baseline (speedup 1.0000x reference)
import jax
import jax.numpy as jnp
from jax.experimental import pallas as pl


def kernel(p0_enc, v0_enc, p0, v0, car_mask, Wc1, Wd1, Wc2, Wd2, Wc3, Wd3):
    raise NotImplementedError("write your pallas kernel here")


# speedup vs baseline: 2.0590x; 2.0590x over previous
import functools

import jax
import jax.numpy as jnp
import numpy as np
from jax.experimental import pallas as pl
from jax.experimental.pallas import tpu as pltpu

B = 32
N = 64
TS = 18
IN_CH = 19
R_ = 3
T_ = 16
K_ = 8
RADIUS = 40.0
DELTA = T_ // K_
TWO_PI = 2.0 * np.pi


def _prep_conv_reg(W):
    O, I = W.shape[0], W.shape[1]
    t_ar = jnp.arange(T_)
    m_ar = jnp.arange(K_)
    s_ar = jnp.arange(K_)
    t_idx = (t_ar[:, None] - m_ar[None, :] * DELTA) % T_
    s_idx = (s_ar[:, None] - m_ar[None, :]) % K_
    Wg = W[:, :, :, t_idx[None, :, :], s_idx[:, None, :]]
    Wz = jnp.transpose(Wg, (1, 3, 2, 4, 0, 5))
    return Wz.reshape(I * K_, R_ * T_ * O * K_)


def _prep_conv_rho1(W):
    O, I = W.shape[0], W.shape[1]
    t_ar = jnp.arange(T_)
    m_ar = jnp.arange(K_)
    t_idx = (t_ar[:, None] - m_ar[None, :] * DELTA) % T_
    W4 = W[:, :, :, t_idx, :]
    phi = TWO_PI * m_ar / K_
    cph, sph = jnp.cos(phi), jnp.sin(phi)
    Rm = jnp.stack([jnp.stack([cph, sph], -1), jnp.stack([-sph, cph], -1)], axis=-2)
    Wz6 = jnp.einsum('oirtms,msu->oirtmu', W4, Rm)
    Wz = jnp.transpose(Wz6, (1, 5, 2, 3, 0, 4))
    return Wz.reshape(I * 2, R_ * T_ * O * K_)


def _prep_lin_reg(Wd):
    O, I = Wd.shape[0], Wd.shape[1]
    k_ar = jnp.arange(K_)
    m_ar = jnp.arange(K_)
    k_idx = (k_ar[:, None] - m_ar[None, :]) % K_
    W4 = Wd[:, :, k_idx]
    return jnp.transpose(W4, (1, 2, 0, 3)).reshape(I * K_, O * K_)


def _prep_lin_lift(Wd):
    O, I = Wd.shape[0], Wd.shape[1]
    m_ar = jnp.arange(K_)
    s_ar = jnp.arange(K_)
    ang = TWO_PI * (m_ar[:, None] + s_ar[None, :]) / K_
    Cc = jnp.einsum('ois,ms->oim', Wd, jnp.cos(ang))
    Cs = jnp.einsum('ois,ms->oim', Wd, jnp.sin(ang))
    Wst = jnp.stack([Cc, Cs], axis=2)
    return jnp.transpose(Wst, (1, 2, 0, 3)).reshape(I * 2, O * K_)


def _scene_kernel(aux_ref, auxt_ref, feats_ref,
                  wz1_ref, wd1_ref, wz2_ref, wd2_ref, wz3_ref, wd3_ref,
                  out_ref):
    f32 = jnp.float32
    px_r = aux_ref[0, 0:1, :]
    py_r = aux_ref[0, 1:2, :]
    mask_r = aux_ref[0, 2:3, :]
    px_c = auxt_ref[0, :, 0:1]
    py_c = auxt_ref[0, :, 1:2]

    relx = px_r - px_c
    rely = py_r - py_c
    n2 = relx * relx + rely * rely
    d = jnp.sqrt(n2 + 1e-12) / RADIUS
    safe = n2 > 1e-12
    rx = jnp.where(safe, relx, 1.0)
    ry = jnp.where(safe, rely, 0.0)
    ang = jnp.arctan2(ry, rx)
    ang = jnp.where(ang < 0.0, ang + TWO_PI, ang)
    wnd = jnp.maximum(1.0 - d * d, 0.0) ** 3
    w = wnd * mask_r
    norm = jnp.sum(w, axis=1, keepdims=True)
    inv_norm = 1.0 / (norm + 1e-8)

    r_pos = jnp.clip(d, 0.0, 1.0) * (R_ - 1)
    r0 = jnp.clip(jnp.floor(r_pos), 0.0, R_ - 2)
    wr = jnp.clip(r_pos - r0, 0.0, 1.0)
    is0 = r0 < 0.5
    A0 = jnp.where(is0, (1.0 - wr) * w, 0.0)
    A1 = jnp.where(is0, wr * w, (1.0 - wr) * w)
    A2 = jnp.where(is0, 0.0, wr * w)
    A = (A0, A1, A2)

    t_pos = ang * (T_ / TWO_PI)
    t0 = jnp.floor(t_pos)
    wt = t_pos - t0
    t0i = jnp.mod(t0.astype(jnp.int32), T_)
    t1i = jnp.mod(t0i + 1, T_)
    toh = []
    one_m_wt = 1.0 - wt
    for t in range(T_):
        toh.append(jnp.where(t0i == t, one_m_wt, 0.0) + jnp.where(t1i == t, wt, 0.0))

    def conv_contract(Z, width):
        acc = jnp.zeros((N, width), f32)
        for r in range(R_):
            Ar = A[r]
            for t in range(T_):
                plane = Ar * toh[t]
                sl = (r * T_ + t) * width
                acc = acc + jnp.dot(plane, Z[:, sl:sl + width],
                                    preferred_element_type=f32)
        return acc

    X1 = feats_ref[0]
    Z1 = jnp.dot(X1, wz1_ref[...], preferred_element_type=f32)
    conv1 = conv_contract(Z1, 64) * inv_norm
    od1 = jnp.dot(X1, wd1_ref[...], preferred_element_type=f32)
    out = jnp.concatenate([conv1, od1], axis=1)

    h = jnp.maximum(out, 0.0)
    Z2 = jnp.dot(h, wz2_ref[...], preferred_element_type=f32)
    conv2 = conv_contract(Z2, 128) * inv_norm
    od2 = jnp.dot(h, wd2_ref[...], preferred_element_type=f32)
    out = conv2 + od2 + out

    h = jnp.maximum(out, 0.0)
    Z3 = jnp.dot(h, wz3_ref[...], preferred_element_type=f32)
    conv3 = conv_contract(Z3, 128) * inv_norm
    od3 = jnp.dot(h, wd3_ref[...], preferred_element_type=f32)
    out = conv3 + od3 + out

    out_ref[0] = jnp.maximum(out, 0.0)


@jax.jit
def kernel(p0_enc, v0_enc, p0, v0, car_mask, Wc1, Wd1, Wc2, Wd2, Wc3, Wd3):
    del p0_enc
    wz1 = _prep_conv_rho1(Wc1)
    wd1 = _prep_lin_lift(Wd1)
    wz2 = _prep_conv_reg(Wc2)
    wd2 = _prep_lin_reg(Wd2)
    wz3 = _prep_conv_reg(Wc3)
    wd3 = _prep_lin_reg(Wd3)

    feats = jnp.concatenate([v0[:, :, None, :], v0_enc], axis=2)
    feats = feats.reshape(B, N, 2 * IN_CH)

    aux = jnp.concatenate([p0[:, :, 0][:, None, :], p0[:, :, 1][:, None, :],
                           jnp.transpose(car_mask, (0, 2, 1)),
                           jnp.zeros((B, 5, N), jnp.float32)], axis=1)
    auxt = jnp.transpose(aux, (0, 2, 1))

    grid = (B,)
    out = pl.pallas_call(
        _scene_kernel,
        grid=grid,
        in_specs=[
            pl.BlockSpec((1, 8, N), lambda b: (b, 0, 0)),
            pl.BlockSpec((1, N, 8), lambda b: (b, 0, 0)),
            pl.BlockSpec((1, N, 2 * IN_CH), lambda b: (b, 0, 0)),
            pl.BlockSpec((2 * IN_CH, R_ * T_ * 64), lambda b: (0, 0)),
            pl.BlockSpec((2 * IN_CH, 64), lambda b: (0, 0)),
            pl.BlockSpec((128, R_ * T_ * 128), lambda b: (0, 0)),
            pl.BlockSpec((128, 128), lambda b: (0, 0)),
            pl.BlockSpec((128, R_ * T_ * 128), lambda b: (0, 0)),
            pl.BlockSpec((128, 128), lambda b: (0, 0)),
        ],
        out_specs=pl.BlockSpec((1, N, 128), lambda b: (b, 0, 0)),
        out_shape=jax.ShapeDtypeStruct((B, N, 128), jnp.float32),
        compiler_params=pltpu.CompilerParams(
            dimension_semantics=("arbitrary",),
        ),
    )(aux, auxt, feats, wz1, wd1, wz2, wd2, wz3, wd3)
    return out.reshape(B, N, 16, K_)
